# baseline (speedup 1.0000x reference)
import jax
import jax.numpy as jnp
from jax.experimental import pallas as pl
from jax.experimental.pallas import tpu as pltpu

D_IN_ = 768
D_HID_ = 3072
D_OUT_ = 768
E_ = 16
NSPLIT_ = 16
C_IN_ = D_IN_ // NSPLIT_
C_HID_ = D_HID_ // NSPLIT_


def _moe_kernel(*refs):
    (xf_ref, wr_ref, br_ref), rest = refs[:3], refs[3:]
    w1_refs = rest[:NSPLIT_]
    b1_ref = rest[NSPLIT_]
    w2_refs = rest[NSPLIT_ + 1:2 * NSPLIT_ + 1]
    b2_ref = rest[2 * NSPLIT_ + 1]
    out_ref = rest[2 * NSPLIT_ + 2]
    route_ref = rest[2 * NSPLIT_ + 3]
    e = pl.program_id(0)

    @pl.when(e == 0)
    def _compute_routing():
        logits = jnp.dot(xf_ref[...], wr_ref[...],
                         preferred_element_type=jnp.float32)
        logits = logits + br_ref[...]
        n, ecnt = logits.shape
        lane = jax.lax.broadcasted_iota(jnp.int32, (n, ecnt), 1)
        neg_inf = jnp.float32(-jnp.inf)
        m1 = jnp.max(logits, axis=1, keepdims=True)
        i1 = jnp.min(jnp.where(logits == m1, lane, ecnt), axis=1, keepdims=True)
        masked = jnp.where(lane == i1, neg_inf, logits)
        m2 = jnp.max(masked, axis=1, keepdims=True)
        i2 = jnp.min(jnp.where(masked == m2, lane, ecnt), axis=1, keepdims=True)
        p1 = 1.0 / (1.0 + jnp.exp(m2 - m1))
        route_ref[:, 0:1] = i1.astype(jnp.float32)
        route_ref[:, 1:2] = i2.astype(jnp.float32)
        route_ref[:, 2:3] = p1
        route_ref[:, 3:4] = 1.0 - p1

    xf = xf_ref[...]
    h = sum(jnp.dot(xf[:, i * C_IN_:(i + 1) * C_IN_], w1_refs[i][0],
                    preferred_element_type=jnp.float32)
            for i in range(NSPLIT_))
    h = jnp.maximum(h + b1_ref[0], 0.0)
    y = sum(jnp.dot(h[:, i * C_HID_:(i + 1) * C_HID_], w2_refs[i][0],
                    preferred_element_type=jnp.float32)
            for i in range(NSPLIT_))
    y = y + b2_ref[0]

    ef = e.astype(jnp.float32)
    gate = (jnp.where(route_ref[:, 0:1] == ef, route_ref[:, 2:3], 0.0)
            + jnp.where(route_ref[:, 1:2] == ef, route_ref[:, 3:4], 0.0))
    contrib = gate * y

    @pl.when(e == 0)
    def _init():
        out_ref[...] = contrib

    @pl.when(e != 0)
    def _acc():
        out_ref[...] += contrib


@jax.jit
def kernel(x, Wr, br, W1, b1, W2, b2):
    Bsz, Ssz, d = x.shape
    xf = x.reshape(-1, d)
    n = xf.shape[0]
    w1_specs = [pl.BlockSpec((1, C_IN_, D_HID_), lambda e, i=i: (e, i, 0))
                for i in range(NSPLIT_)]
    w2_specs = [pl.BlockSpec((1, C_HID_, D_OUT_), lambda e, i=i: (e, i, 0))
                for i in range(NSPLIT_)]
    out = pl.pallas_call(
        _moe_kernel,
        grid=(E_,),
        in_specs=[
            pl.BlockSpec((n, D_IN_), lambda e: (0, 0)),
            pl.BlockSpec((D_IN_, E_), lambda e: (0, 0)),
            pl.BlockSpec((1, E_), lambda e: (0, 0)),
        ] + w1_specs + [
            pl.BlockSpec((1, 1, D_HID_), lambda e: (e, 0, 0)),
        ] + w2_specs + [
            pl.BlockSpec((1, 1, D_OUT_), lambda e: (e, 0, 0)),
        ],
        out_specs=pl.BlockSpec((n, D_OUT_), lambda e: (0, 0)),
        out_shape=jax.ShapeDtypeStruct((n, D_OUT_), jnp.float32),
        scratch_shapes=[pltpu.VMEM((n, 8), jnp.float32)],
    )(xf, Wr, br.reshape(1, E_), *([W1] * NSPLIT_),
      b1.reshape(E_, 1, D_HID_), *([W2] * NSPLIT_),
      b2.reshape(E_, 1, D_OUT_))
    return out.reshape(Bsz, Ssz, D_OUT_)


# speedup vs baseline: 1.0551x; 1.0551x over previous
import jax
import jax.numpy as jnp
from jax.experimental import pallas as pl
from jax.experimental.pallas import tpu as pltpu

D_IN_ = 768
D_HID_ = 3072
D_OUT_ = 768
E_ = 16
NSPLIT_ = 8
C_IN_ = D_IN_ // NSPLIT_
C_HID_ = D_HID_ // NSPLIT_


def _moe_kernel(*refs):
    (xf_ref, wr_ref, br_ref), rest = refs[:3], refs[3:]
    w1_refs = rest[:NSPLIT_]
    b1_ref = rest[NSPLIT_]
    w2_refs = rest[NSPLIT_ + 1:2 * NSPLIT_ + 1]
    b2_ref = rest[2 * NSPLIT_ + 1]
    out_ref = rest[2 * NSPLIT_ + 2]
    route_ref = rest[2 * NSPLIT_ + 3]
    e = pl.program_id(0)

    @pl.when(e == 0)
    def _compute_routing():
        logits = jnp.dot(xf_ref[...], wr_ref[...],
                         preferred_element_type=jnp.float32)
        logits = logits + br_ref[...]
        n, ecnt = logits.shape
        lane = jax.lax.broadcasted_iota(jnp.int32, (n, ecnt), 1)
        neg_inf = jnp.float32(-jnp.inf)
        m1 = jnp.max(logits, axis=1, keepdims=True)
        i1 = jnp.min(jnp.where(logits == m1, lane, ecnt), axis=1, keepdims=True)
        masked = jnp.where(lane == i1, neg_inf, logits)
        m2 = jnp.max(masked, axis=1, keepdims=True)
        i2 = jnp.min(jnp.where(masked == m2, lane, ecnt), axis=1, keepdims=True)
        p1 = 1.0 / (1.0 + jnp.exp(m2 - m1))
        route_ref[:, 0:1] = i1.astype(jnp.float32)
        route_ref[:, 1:2] = i2.astype(jnp.float32)
        route_ref[:, 2:3] = p1
        route_ref[:, 3:4] = 1.0 - p1

    xf = xf_ref[...]
    h = sum(jnp.dot(xf[:, i * C_IN_:(i + 1) * C_IN_], w1_refs[i][0],
                    preferred_element_type=jnp.float32)
            for i in range(NSPLIT_))
    h = jnp.maximum(h + b1_ref[0], 0.0)
    y = sum(jnp.dot(h[:, i * C_HID_:(i + 1) * C_HID_], w2_refs[i][0],
                    preferred_element_type=jnp.float32)
            for i in range(NSPLIT_))
    y = y + b2_ref[0]

    ef = e.astype(jnp.float32)
    gate = (jnp.where(route_ref[:, 0:1] == ef, route_ref[:, 2:3], 0.0)
            + jnp.where(route_ref[:, 1:2] == ef, route_ref[:, 3:4], 0.0))
    contrib = gate * y

    @pl.when(e == 0)
    def _init():
        out_ref[...] = contrib

    @pl.when(e != 0)
    def _acc():
        out_ref[...] += contrib


@jax.jit
def kernel(x, Wr, br, W1, b1, W2, b2):
    Bsz, Ssz, d = x.shape
    xf = x.reshape(-1, d)
    n = xf.shape[0]
    w1_specs = [pl.BlockSpec((1, C_IN_, D_HID_), lambda e, i=i: (e, i, 0))
                for i in range(NSPLIT_)]
    w2_specs = [pl.BlockSpec((1, C_HID_, D_OUT_), lambda e, i=i: (e, i, 0))
                for i in range(NSPLIT_)]
    out = pl.pallas_call(
        _moe_kernel,
        grid=(E_,),
        in_specs=[
            pl.BlockSpec((n, D_IN_), lambda e: (0, 0)),
            pl.BlockSpec((D_IN_, E_), lambda e: (0, 0)),
            pl.BlockSpec((1, E_), lambda e: (0, 0)),
        ] + w1_specs + [
            pl.BlockSpec((1, 1, D_HID_), lambda e: (e, 0, 0)),
        ] + w2_specs + [
            pl.BlockSpec((1, 1, D_OUT_), lambda e: (e, 0, 0)),
        ],
        out_specs=pl.BlockSpec((n, D_OUT_), lambda e: (0, 0)),
        out_shape=jax.ShapeDtypeStruct((n, D_OUT_), jnp.float32),
        scratch_shapes=[pltpu.VMEM((n, 8), jnp.float32)],
    )(xf, Wr, br.reshape(1, E_), *([W1] * NSPLIT_),
      b1.reshape(E_, 1, D_HID_), *([W2] * NSPLIT_),
      b2.reshape(E_, 1, D_OUT_))
    return out.reshape(Bsz, Ssz, D_OUT_)
